# Initial kernel scaffold; baseline (speedup 1.0000x reference)
import jax
import jax.numpy as jnp
from jax.experimental import pallas as pl


def kernel(x, up_row, up_col, up_value, row_map, indices, w_dw, b_dw, w_pw, b_pw):
    raise NotImplementedError("write your pallas kernel here")


# speedup vs baseline: 1.4621x; 1.4621x over previous
import functools

import jax
import jax.numpy as jnp
from jax import lax
from jax.experimental import pallas as pl
from jax.experimental.pallas import tpu as pltpu
from jax.experimental.pallas import tpu_sc as plsc

NC, NS = 2, 16
NW = NC * NS
L = 16

N_IN = 12500
N_OUT = 50000
C = 128
S = 9
NCK = C // L

N_PAD = 51200
NV_W = N_PAD // NW

NB1 = 40
NBATCH1 = NV_W // NB1
NB2 = 32
NBATCH2 = NV_W // NB2

_mesh = plsc.VectorSubcoreMesh(core_axis_name="c", subcore_axis_name="s")


def _worker_id():
    return lax.axis_index("s") * NC + lax.axis_index("c")


def _pool_body(x_hbm, cols_hbm, vals_hbm, out_hbm, idx_v, vals_v, rows_v, out_v, sem):
    v0 = _worker_id() * NV_W

    def batch(b, carry):
        row0 = v0 + b * NB1
        off = row0 * 3
        pltpu.sync_copy(cols_hbm.at[pl.ds(off, 3 * NB1)], idx_v)
        pltpu.sync_copy(vals_hbm.at[pl.ds(off, 3 * NB1)], vals_v)
        pltpu.async_copy(x_hbm.at[idx_v], rows_v, sem).wait()

        def vert(i, c2):
            w0 = vals_v[3 * i, :]
            w1 = vals_v[3 * i + 1, :]
            w2 = vals_v[3 * i + 2, :]
            for k in range(NCK):
                acc = rows_v[3 * i, pl.ds(L * k, L)] * w0
                acc = acc + rows_v[3 * i + 1, pl.ds(L * k, L)] * w1
                acc = acc + rows_v[3 * i + 2, pl.ds(L * k, L)] * w2
                out_v[i, pl.ds(L * k, L)] = acc
            return c2

        lax.fori_loop(0, NB1, vert, 0)
        pltpu.sync_copy(out_v, out_hbm.at[pl.ds(row0, NB1)])
        return carry

    lax.fori_loop(0, NBATCH1, batch, 0)


@functools.partial(
    pl.kernel,
    out_type=jax.ShapeDtypeStruct((N_PAD, C), jnp.float32),
    mesh=_mesh,
    scratch_types=[
        pltpu.VMEM((3 * NB1,), jnp.int32),
        pltpu.VMEM((3 * NB1, L), jnp.float32),
        pltpu.VMEM((3 * NB1, C), jnp.float32),
        pltpu.VMEM((NB1, C), jnp.float32),
        pltpu.SemaphoreType.DMA,
    ],
)
def _pool_kernel(x_hbm, cols_hbm, vals_hbm, out_hbm, idx_v, vals_v, rows_v, out_v, sem):
    _pool_body(x_hbm, cols_hbm, vals_hbm, out_hbm, idx_v, vals_v, rows_v, out_v, sem)


def _dw_body(pooled_hbm, sidx_hbm, wdw_hbm, out_hbm, idx_v, rows_v, out_v, wdw_v, sem):
    v0 = _worker_id() * NV_W
    pltpu.sync_copy(wdw_hbm, wdw_v)

    def batch(b, carry):
        row0 = v0 + b * NB2
        off = row0 * S
        for t in range(3):
            pltpu.sync_copy(sidx_hbm.at[pl.ds(off + 96 * t, 96)], idx_v.at[t])
        cps = [
            pltpu.async_copy(pooled_hbm.at[idx_v.at[t]],
                             rows_v.at[pl.ds(96 * t, 96)], sem)
            for t in range(3)
        ]
        for cp in cps:
            cp.wait()

        for k in range(NCK):
            w = [wdw_v[s, pl.ds(L * k, L)] for s in range(S)]

            def vert(i, c2):
                acc = rows_v[S * i, pl.ds(L * k, L)] * w[0]
                for s in range(1, S):
                    acc = acc + rows_v[S * i + s, pl.ds(L * k, L)] * w[s]
                out_v[i, pl.ds(L * k, L)] = acc
                return c2

            lax.fori_loop(0, NB2, vert, 0)
        pltpu.sync_copy(out_v, out_hbm.at[pl.ds(row0, NB2)])
        return carry

    lax.fori_loop(0, NBATCH2, batch, 0)


@functools.partial(
    pl.kernel,
    out_type=jax.ShapeDtypeStruct((N_PAD, C), jnp.float32),
    mesh=_mesh,
    scratch_types=[
        pltpu.VMEM((3, 96), jnp.int32),
        pltpu.VMEM((S * NB2, C), jnp.float32),
        pltpu.VMEM((NB2, C), jnp.float32),
        pltpu.VMEM((S, C), jnp.float32),
        pltpu.SemaphoreType.DMA,
    ],
)
def _dw_kernel(pooled_hbm, sidx_hbm, wdw_hbm, out_hbm, idx_v, rows_v, out_v, wdw_v, sem):
    _dw_body(pooled_hbm, sidx_hbm, wdw_hbm, out_hbm, idx_v, rows_v, out_v, wdw_v, sem)


MM_BLK = 512


def _mm_body(dw_ref, wpw_ref, bdw_ref, bpw_ref, o_ref):
    a = dw_ref[...] + bdw_ref[...]
    acc = jnp.dot(a, wpw_ref[...], preferred_element_type=jnp.float32)
    o_ref[...] = jnp.maximum(acc + bpw_ref[...], 0.0)


def _pointwise(dw, w_pw, b_dw, b_pw):
    return pl.pallas_call(
        _mm_body,
        grid=(N_PAD // MM_BLK,),
        in_specs=[
            pl.BlockSpec((MM_BLK, C), lambda b: (b, 0)),
            pl.BlockSpec((C, C), lambda b: (0, 0)),
            pl.BlockSpec((1, C), lambda b: (0, 0)),
            pl.BlockSpec((1, C), lambda b: (0, 0)),
        ],
        out_specs=pl.BlockSpec((MM_BLK, C), lambda b: (b, 0)),
        out_shape=jax.ShapeDtypeStruct((N_PAD, C), jnp.float32),
    )(dw, w_pw, b_dw.reshape(1, C), b_pw.reshape(1, C))


def kernel(x, up_row, up_col, up_value, row_map, indices, w_dw, b_dw, w_pw, b_pw):
    del up_row
    x2 = x.reshape(N_IN, C).astype(jnp.float32)
    rm = row_map.astype(jnp.int32)
    cols3 = jnp.take(up_col.astype(jnp.int32), rm, axis=0)
    vals3 = jnp.take(up_value.astype(jnp.float32), rm, axis=0)
    pad = N_PAD - N_OUT
    cols_flat = jnp.pad(cols3, ((0, pad), (0, 0))).reshape(-1)
    vals_flat = jnp.pad(vals3, ((0, pad), (0, 0))).reshape(-1)
    vals16 = jnp.repeat(vals_flat[:, None], L, axis=1)
    sidx_flat = jnp.pad(indices.astype(jnp.int32), ((0, pad), (0, 0))).reshape(-1)
    wdw_t = w_dw.astype(jnp.float32).T

    pooled = _pool_kernel(x2, cols_flat, vals16)
    dw = _dw_kernel(pooled, sidx_flat, wdw_t)
    pw = _pointwise(dw, w_pw.astype(jnp.float32), b_dw, b_pw)
    return pw[:N_OUT].reshape(1, N_OUT, C)


# speedup vs baseline: 1.8850x; 1.2893x over previous
import functools

import jax
import jax.numpy as jnp
from jax import lax
from jax.experimental import pallas as pl
from jax.experimental.pallas import tpu as pltpu
from jax.experimental.pallas import tpu_sc as plsc

NC, NS = 2, 16
NW = NC * NS
L = 16

N_IN = 12500
N_OUT = 50000
C = 128
S = 9
NCK = C // L

N_PAD = 51200
NV_W = N_PAD // NW

NB1 = 40
NBATCH1 = NV_W // NB1
NB2 = 32
NBATCH2 = NV_W // NB2

_mesh = plsc.VectorSubcoreMesh(core_axis_name="c", subcore_axis_name="s")


def _worker_id():
    return lax.axis_index("s") * NC + lax.axis_index("c")


def _pool_compute(vals_v, rows_v, out_v):
    def vert(ii, c2):
        for u in range(2):
            i = 2 * ii + u
            w0 = vals_v[3 * i, :]
            w1 = vals_v[3 * i + 1, :]
            w2 = vals_v[3 * i + 2, :]
            for k in range(NCK):
                acc = rows_v[3 * i, pl.ds(L * k, L)] * w0
                acc = acc + rows_v[3 * i + 1, pl.ds(L * k, L)] * w1
                acc = acc + rows_v[3 * i + 2, pl.ds(L * k, L)] * w2
                out_v[i, pl.ds(L * k, L)] = acc
        return c2

    lax.fori_loop(0, NB1 // 2, vert, 0)


def _pool_body(x_hbm, cols_hbm, vals_hbm, out_hbm,
               cols_v, vals_v, rows_v, out_v, gsem, vsem, osem):
    v0 = _worker_id() * NV_W
    pltpu.sync_copy(cols_hbm.at[pl.ds(v0 * 3, 3 * NB1 * NBATCH1)], cols_v)

    def fire(b, slot):
        pltpu.async_copy(vals_hbm.at[pl.ds((v0 + b * NB1) * 3, 3 * NB1)],
                         vals_v.at[slot], vsem.at[slot])
        pltpu.async_copy(x_hbm.at[cols_v.at[pl.ds(b * 3 * NB1, 3 * NB1)]],
                         rows_v.at[slot], gsem.at[slot])

    def wait_in(slot):
        pltpu.make_async_copy(vals_hbm.at[pl.ds(0, 3 * NB1)],
                              vals_v.at[slot], vsem.at[slot]).wait()
        pltpu.make_async_copy(x_hbm.at[pl.ds(0, 3 * NB1)],
                              rows_v.at[slot], gsem.at[slot]).wait()

    def wait_out(slot):
        pltpu.make_async_copy(out_v.at[slot],
                              out_hbm.at[pl.ds(0, NB1)], osem.at[slot]).wait()

    fire(0, 0)

    def step(b2, carry):
        b = 2 * b2
        wait_in(0)
        fire(b + 1, 1)

        @pl.when(b2 > 0)
        def _():
            wait_out(0)

        _pool_compute(vals_v.at[0], rows_v.at[0], out_v.at[0])
        pltpu.async_copy(out_v.at[0], out_hbm.at[pl.ds(v0 + b * NB1, NB1)],
                         osem.at[0])
        wait_in(1)

        @pl.when(b2 < NBATCH1 // 2 - 1)
        def _():
            fire(b + 2, 0)

        @pl.when(b2 > 0)
        def _():
            wait_out(1)

        _pool_compute(vals_v.at[1], rows_v.at[1], out_v.at[1])
        pltpu.async_copy(out_v.at[1],
                         out_hbm.at[pl.ds(v0 + (b + 1) * NB1, NB1)],
                         osem.at[1])
        return carry

    lax.fori_loop(0, NBATCH1 // 2, step, 0)
    wait_out(0)
    wait_out(1)


@functools.partial(
    pl.kernel,
    out_type=jax.ShapeDtypeStruct((N_PAD, C), jnp.float32),
    mesh=_mesh,
    scratch_types=[
        pltpu.VMEM((3 * NB1 * NBATCH1,), jnp.int32),
        pltpu.VMEM((2, 3 * NB1, L), jnp.float32),
        pltpu.VMEM((2, 3 * NB1, C), jnp.float32),
        pltpu.VMEM((2, NB1, C), jnp.float32),
        pltpu.SemaphoreType.DMA((2,)),
        pltpu.SemaphoreType.DMA((2,)),
        pltpu.SemaphoreType.DMA((2,)),
    ],
)
def _pool_kernel(x_hbm, cols_hbm, vals_hbm, out_hbm,
                 cols_v, vals_v, rows_v, out_v, gsem, vsem, osem):
    _pool_body(x_hbm, cols_hbm, vals_hbm, out_hbm,
               cols_v, vals_v, rows_v, out_v, gsem, vsem, osem)


def _dw_compute(wdw_v, rows_v, out_v):
    for k in range(NCK):
        w = [wdw_v[s, pl.ds(L * k, L)] for s in range(S)]

        def vert(ii, c2):
            for u in range(2):
                i = 2 * ii + u
                acc = rows_v[S * i, pl.ds(L * k, L)] * w[0]
                for s in range(1, S):
                    acc = acc + rows_v[S * i + s, pl.ds(L * k, L)] * w[s]
                out_v[i, pl.ds(L * k, L)] = acc
            return c2

        lax.fori_loop(0, NB2 // 2, vert, 0)


def _dw_body(pooled_hbm, sidx_hbm, wdw_hbm, out_hbm,
             sidx_v, rows_v, out_v, wdw_v, gsem, osem):
    v0 = _worker_id() * NV_W
    pltpu.sync_copy(wdw_hbm, wdw_v)
    pltpu.sync_copy(sidx_hbm.at[pl.ds(v0 * S, S * NB2 * NBATCH2)], sidx_v)

    def fire(b, slot):
        for t in range(3):
            pltpu.async_copy(
                pooled_hbm.at[sidx_v.at[pl.ds(b * S * NB2 + 96 * t, 96)]],
                rows_v.at[slot, pl.ds(96 * t, 96)], gsem.at[slot])

    def wait_in(slot):
        pltpu.make_async_copy(pooled_hbm.at[pl.ds(0, S * NB2)],
                              rows_v.at[slot], gsem.at[slot]).wait()

    def wait_out(slot):
        pltpu.make_async_copy(out_v.at[slot],
                              out_hbm.at[pl.ds(0, NB2)], osem.at[slot]).wait()

    fire(0, 0)

    def step(b2, carry):
        b = 2 * b2
        wait_in(0)
        fire(b + 1, 1)

        @pl.when(b2 > 0)
        def _():
            wait_out(0)

        _dw_compute(wdw_v, rows_v.at[0], out_v.at[0])
        pltpu.async_copy(out_v.at[0], out_hbm.at[pl.ds(v0 + b * NB2, NB2)],
                         osem.at[0])

        wait_in(1)

        @pl.when(b2 < NBATCH2 // 2 - 1)
        def _():
            fire(b + 2, 0)

        @pl.when(b2 > 0)
        def _():
            wait_out(1)

        _dw_compute(wdw_v, rows_v.at[1], out_v.at[1])
        pltpu.async_copy(out_v.at[1],
                         out_hbm.at[pl.ds(v0 + (b + 1) * NB2, NB2)],
                         osem.at[1])
        return carry

    lax.fori_loop(0, NBATCH2 // 2, step, 0)
    wait_out(0)
    wait_out(1)


@functools.partial(
    pl.kernel,
    out_type=jax.ShapeDtypeStruct((N_PAD, C), jnp.float32),
    mesh=_mesh,
    scratch_types=[
        pltpu.VMEM((S * NB2 * NBATCH2,), jnp.int32),
        pltpu.VMEM((2, S * NB2, C), jnp.float32),
        pltpu.VMEM((2, NB2, C), jnp.float32),
        pltpu.VMEM((S, C), jnp.float32),
        pltpu.SemaphoreType.DMA((2,)),
        pltpu.SemaphoreType.DMA((2,)),
    ],
)
def _dw_kernel(pooled_hbm, sidx_hbm, wdw_hbm, out_hbm,
               sidx_v, rows_v, out_v, wdw_v, gsem, osem):
    _dw_body(pooled_hbm, sidx_hbm, wdw_hbm, out_hbm,
             sidx_v, rows_v, out_v, wdw_v, gsem, osem)


MM_BLK = 512


def _mm_body(dw_ref, wpw_ref, bdw_ref, bpw_ref, o_ref):
    a = dw_ref[...] + bdw_ref[...]
    acc = jnp.dot(a, wpw_ref[...], preferred_element_type=jnp.float32)
    o_ref[...] = jnp.maximum(acc + bpw_ref[...], 0.0)


def _pointwise(dw, w_pw, b_dw, b_pw):
    return pl.pallas_call(
        _mm_body,
        grid=(N_PAD // MM_BLK,),
        in_specs=[
            pl.BlockSpec((MM_BLK, C), lambda b: (b, 0)),
            pl.BlockSpec((C, C), lambda b: (0, 0)),
            pl.BlockSpec((1, C), lambda b: (0, 0)),
            pl.BlockSpec((1, C), lambda b: (0, 0)),
        ],
        out_specs=pl.BlockSpec((MM_BLK, C), lambda b: (b, 0)),
        out_shape=jax.ShapeDtypeStruct((N_PAD, C), jnp.float32),
    )(dw, w_pw, b_dw.reshape(1, C), b_pw.reshape(1, C))


def kernel(x, up_row, up_col, up_value, row_map, indices, w_dw, b_dw, w_pw, b_pw):
    del up_row
    x2 = x.reshape(N_IN, C).astype(jnp.float32)
    rm = row_map.astype(jnp.int32)
    cols3 = jnp.take(up_col.astype(jnp.int32), rm, axis=0)
    vals3 = jnp.take(up_value.astype(jnp.float32), rm, axis=0)
    pad = N_PAD - N_OUT
    cols_flat = jnp.pad(cols3, ((0, pad), (0, 0))).reshape(-1)
    vals_flat = jnp.pad(vals3, ((0, pad), (0, 0))).reshape(-1)
    vals16 = jnp.repeat(vals_flat[:, None], L, axis=1)
    sidx_flat = jnp.pad(indices.astype(jnp.int32), ((0, pad), (0, 0))).reshape(-1)
    wdw_t = w_dw.astype(jnp.float32).T

    pooled = _pool_kernel(x2, cols_flat, vals16)
    dw = _dw_kernel(pooled, sidx_flat, wdw_t)
    pw = _pointwise(dw, w_pw.astype(jnp.float32), b_dw, b_pw)
    return pw[:N_OUT].reshape(1, N_OUT, C)


# speedup vs baseline: 2.0030x; 1.0626x over previous
import functools

import jax
import jax.numpy as jnp
from jax import lax
from jax.experimental import pallas as pl
from jax.experimental.pallas import tpu as pltpu
from jax.experimental.pallas import tpu_sc as plsc

NC, NS = 2, 16
NW = NC * NS
L = 16

N_IN = 12500
N_OUT = 50000
C = 128
S = 9
NCK = C // L

N_PAD = 51200
NV_W = N_PAD // NW

NB1 = 40
NBATCH1 = NV_W // NB1
NB2 = 32
NBATCH2 = NV_W // NB2

_mesh = plsc.VectorSubcoreMesh(core_axis_name="c", subcore_axis_name="s")


def _worker_id():
    return lax.axis_index("s") * NC + lax.axis_index("c")


_GDN = lax.GatherDimensionNumbers(
    offset_dims=(), collapsed_slice_dims=(0,), start_index_map=(0,))


def _bcast(vals_ref, p):
    win = vals_ref[pl.ds((p // L) * L, L)]
    idx = jnp.full((L, 1), p % L, jnp.int32)
    return lax.gather(win, idx, _GDN, (1,),
                      mode=lax.GatherScatterMode.PROMISE_IN_BOUNDS)


def _pool_compute(vals_v, rows_v, out_v):
    def vert(ii, c2):
        for u in range(2):
            i = 2 * ii + u
            w0 = _bcast(vals_v, 3 * i)
            w1 = _bcast(vals_v, 3 * i + 1)
            w2 = _bcast(vals_v, 3 * i + 2)
            for k in range(NCK):
                acc = rows_v[3 * i, pl.ds(L * k, L)] * w0
                acc = acc + rows_v[3 * i + 1, pl.ds(L * k, L)] * w1
                acc = acc + rows_v[3 * i + 2, pl.ds(L * k, L)] * w2
                out_v[i, pl.ds(L * k, L)] = acc
        return c2

    lax.fori_loop(0, NB1 // 2, vert, 0)


def _pool_body(x_hbm, rm_hbm, ucol_hbm, uval_hbm, out_hbm,
               rm_v, cols_v, vals_v, rows_v, out_v, csem, gsem, osem):
    v0 = _worker_id() * NV_W
    pltpu.sync_copy(rm_hbm.at[pl.ds(v0 * 3, 3 * NB1 * NBATCH1)], rm_v)

    def fire_cv(b, slot):
        idx = rm_v.at[pl.ds(b * 3 * NB1, 3 * NB1)]
        pltpu.async_copy(ucol_hbm.at[idx], cols_v.at[slot], csem.at[slot])
        pltpu.async_copy(uval_hbm.at[idx], vals_v.at[slot], csem.at[slot])

    def wait_cv(slot):
        pltpu.make_async_copy(ucol_hbm.at[pl.ds(0, 3 * NB1)],
                              cols_v.at[slot], csem.at[slot]).wait()
        pltpu.make_async_copy(uval_hbm.at[pl.ds(0, 3 * NB1)],
                              vals_v.at[slot], csem.at[slot]).wait()

    def fire_rows(slot):
        pltpu.async_copy(x_hbm.at[cols_v.at[slot]], rows_v.at[slot],
                         gsem.at[slot])

    def wait_rows(slot):
        pltpu.make_async_copy(x_hbm.at[pl.ds(0, 3 * NB1)],
                              rows_v.at[slot], gsem.at[slot]).wait()

    def wait_out(slot):
        pltpu.make_async_copy(out_v.at[slot],
                              out_hbm.at[pl.ds(0, NB1)], osem.at[slot]).wait()

    fire_cv(0, 0)
    fire_cv(1, 1)
    wait_cv(0)
    fire_rows(0)

    def step(b2, carry):
        b = 2 * b2
        for slot in (0, 1):
            bb = b + slot
            wait_rows(slot)
            @pl.when(bb + 1 < NBATCH1)
            def _():
                wait_cv(1 - slot)
                fire_rows(1 - slot)

            @pl.when(b2 > 0)
            def _():
                wait_out(slot)

            _pool_compute(vals_v.at[slot], rows_v.at[slot], out_v.at[slot])

            @pl.when(bb + 2 < NBATCH1)
            def _():
                fire_cv(bb + 2, slot)

            pltpu.async_copy(out_v.at[slot],
                             out_hbm.at[pl.ds(v0 + bb * NB1, NB1)],
                             osem.at[slot])
        return carry

    lax.fori_loop(0, NBATCH1 // 2, step, 0)
    wait_out(0)
    wait_out(1)


@functools.partial(
    pl.kernel,
    out_type=jax.ShapeDtypeStruct((N_PAD, C), jnp.float32),
    mesh=_mesh,
    scratch_types=[
        pltpu.VMEM((3 * NB1 * NBATCH1,), jnp.int32),
        pltpu.VMEM((2, 3 * NB1), jnp.int32),
        pltpu.VMEM((2, 3 * NB1), jnp.float32),
        pltpu.VMEM((2, 3 * NB1, C), jnp.float32),
        pltpu.VMEM((2, NB1, C), jnp.float32),
        pltpu.SemaphoreType.DMA((2,)),
        pltpu.SemaphoreType.DMA((2,)),
        pltpu.SemaphoreType.DMA((2,)),
    ],
)
def _pool_kernel(x_hbm, rm_hbm, ucol_hbm, uval_hbm, out_hbm,
                 rm_v, cols_v, vals_v, rows_v, out_v, csem, gsem, osem):
    _pool_body(x_hbm, rm_hbm, ucol_hbm, uval_hbm, out_hbm,
               rm_v, cols_v, vals_v, rows_v, out_v, csem, gsem, osem)


def _dw_compute(wdw_v, rows_v, out_v):
    for k in range(NCK):
        w = [wdw_v[s, pl.ds(L * k, L)] for s in range(S)]

        def vert(ii, c2):
            for u in range(2):
                i = 2 * ii + u
                acc = rows_v[S * i, pl.ds(L * k, L)] * w[0]
                for s in range(1, S):
                    acc = acc + rows_v[S * i + s, pl.ds(L * k, L)] * w[s]
                out_v[i, pl.ds(L * k, L)] = acc
            return c2

        lax.fori_loop(0, NB2 // 2, vert, 0)


def _dw_body(pooled_hbm, sidx_hbm, wdw_hbm, out_hbm,
             sidx_v, rows_v, out_v, wdw_v, gsem, osem):
    v0 = _worker_id() * NV_W
    pltpu.sync_copy(wdw_hbm, wdw_v)
    pltpu.sync_copy(sidx_hbm.at[pl.ds(v0 * S, S * NB2 * NBATCH2)], sidx_v)

    def fire(b, slot):
        for t in range(3):
            pltpu.async_copy(
                pooled_hbm.at[sidx_v.at[pl.ds(b * S * NB2 + 96 * t, 96)]],
                rows_v.at[slot, pl.ds(96 * t, 96)], gsem.at[slot])

    def wait_in(slot):
        pltpu.make_async_copy(pooled_hbm.at[pl.ds(0, S * NB2)],
                              rows_v.at[slot], gsem.at[slot]).wait()

    def wait_out(slot):
        pltpu.make_async_copy(out_v.at[slot],
                              out_hbm.at[pl.ds(0, NB2)], osem.at[slot]).wait()

    fire(0, 0)

    def step(b2, carry):
        b = 2 * b2
        for slot in (0, 1):
            bb = b + slot
            wait_in(slot)

            @pl.when(bb + 1 < NBATCH2)
            def _():
                fire(bb + 1, 1 - slot)

            @pl.when(b2 > 0)
            def _():
                wait_out(slot)

            _dw_compute(wdw_v, rows_v.at[slot], out_v.at[slot])
            pltpu.async_copy(out_v.at[slot],
                             out_hbm.at[pl.ds(v0 + bb * NB2, NB2)],
                             osem.at[slot])
        return carry

    lax.fori_loop(0, NBATCH2 // 2, step, 0)
    wait_out(0)
    wait_out(1)


@functools.partial(
    pl.kernel,
    out_type=jax.ShapeDtypeStruct((N_PAD, C), jnp.float32),
    mesh=_mesh,
    scratch_types=[
        pltpu.VMEM((S * NB2 * NBATCH2,), jnp.int32),
        pltpu.VMEM((2, S * NB2, C), jnp.float32),
        pltpu.VMEM((2, NB2, C), jnp.float32),
        pltpu.VMEM((S, C), jnp.float32),
        pltpu.SemaphoreType.DMA((2,)),
        pltpu.SemaphoreType.DMA((2,)),
    ],
)
def _dw_kernel(pooled_hbm, sidx_hbm, wdw_hbm, out_hbm,
               sidx_v, rows_v, out_v, wdw_v, gsem, osem):
    _dw_body(pooled_hbm, sidx_hbm, wdw_hbm, out_hbm,
             sidx_v, rows_v, out_v, wdw_v, gsem, osem)


MM_BLK = 400


def _mm_body(dw_ref, wpw_ref, bdw_ref, bpw_ref, o_ref):
    a = dw_ref[...] + bdw_ref[...]
    acc = jnp.dot(a, wpw_ref[...], preferred_element_type=jnp.float32)
    o_ref[...] = jnp.maximum(acc + bpw_ref[...], 0.0)


def _pointwise(dw, w_pw, b_dw, b_pw):
    return pl.pallas_call(
        _mm_body,
        grid=(N_OUT // MM_BLK,),
        in_specs=[
            pl.BlockSpec((MM_BLK, C), lambda b: (b, 0)),
            pl.BlockSpec((C, C), lambda b: (0, 0)),
            pl.BlockSpec((1, C), lambda b: (0, 0)),
            pl.BlockSpec((1, C), lambda b: (0, 0)),
        ],
        out_specs=pl.BlockSpec((MM_BLK, C), lambda b: (b, 0)),
        out_shape=jax.ShapeDtypeStruct((N_OUT, C), jnp.float32),
    )(dw, w_pw, b_dw.reshape(1, C), b_pw.reshape(1, C))


def kernel(x, up_row, up_col, up_value, row_map, indices, w_dw, b_dw, w_pw, b_pw):
    del up_row
    x2 = x.reshape(N_IN, C)
    pad = N_PAD - N_OUT
    rm_flat = jnp.pad(row_map.astype(jnp.int32), ((0, pad), (0, 0))).reshape(-1)
    sidx_flat = jnp.pad(indices.astype(jnp.int32), ((0, pad), (0, 0))).reshape(-1)
    wdw_t = w_dw.T

    pooled = _pool_kernel(x2, rm_flat, up_col.astype(jnp.int32), up_value)
    dw = _dw_kernel(pooled, sidx_flat, wdw_t)
    pw = _pointwise(dw, w_pw, b_dw, b_pw)
    return pw.reshape(1, N_OUT, C)


# speedup vs baseline: 3.6917x; 1.8430x over previous
import functools

import jax
import jax.numpy as jnp
from jax import lax
from jax.experimental import pallas as pl
from jax.experimental.pallas import tpu as pltpu
from jax.experimental.pallas import tpu_sc as plsc

NC, NS = 2, 16
NW = NC * NS
L = 16

N_IN = 12500
N_OUT = 50000
C = 128
S = 9
NCK = C // L

NB1 = 40
P1, Q1 = 48, 32
COV1 = NS * (P1 + Q1) * NB1
TAIL1 = Q1 - (COV1 - N_OUT) // NB1

NB2 = 32
P2, Q2 = 76, 22
N_PAD2 = NS * (P2 + Q2) * NB2

_mesh = plsc.VectorSubcoreMesh(core_axis_name="c", subcore_axis_name="s")


_GDN = lax.GatherDimensionNumbers(
    offset_dims=(), collapsed_slice_dims=(0,), start_index_map=(0,))


def _bcast(vals_ref, p):
    win = vals_ref[pl.ds((p // L) * L, L)]
    idx = jnp.full((L, 1), p % L, jnp.int32)
    return lax.gather(win, idx, _GDN, (1,),
                      mode=lax.GatherScatterMode.PROMISE_IN_BOUNDS)


def _pool_compute(vals_v, rows_v, out_v):
    def vert(ii, c2):
        for u in range(2):
            i = 2 * ii + u
            w0 = _bcast(vals_v, 3 * i)
            w1 = _bcast(vals_v, 3 * i + 1)
            w2 = _bcast(vals_v, 3 * i + 2)
            for k in range(NCK):
                acc = rows_v[3 * i, pl.ds(L * k, L)] * w0
                acc = acc + rows_v[3 * i + 1, pl.ds(L * k, L)] * w1
                acc = acc + rows_v[3 * i + 2, pl.ds(L * k, L)] * w2
                out_v[i, pl.ds(L * k, L)] = acc
        return c2

    lax.fori_loop(0, NB1 // 2, vert, 0)


def _pool_body(x_hbm, rm_hbm, ucol_hbm, uval_hbm, out_hbm,
               rm_v, cols_v, vals_v, rows_v, out_v, csem, gsem, osem):
    s = lax.axis_index("s")
    c = lax.axis_index("c")
    b0 = s * (P1 + Q1) + c * P1
    v0 = b0 * NB1
    last = jnp.logical_and(s == NS - 1, c == 1)
    nb = jnp.where(last, TAIL1, jnp.where(c == 0, P1, Q1))

    @pl.when(jnp.logical_and(c == 0, jnp.logical_not(last)))
    def _():
        pltpu.sync_copy(rm_hbm.at[pl.ds(v0 * 3, 3 * NB1 * P1)],
                        rm_v.at[pl.ds(0, 3 * NB1 * P1)])

    @pl.when(jnp.logical_and(c == 1, jnp.logical_not(last)))
    def _():
        pltpu.sync_copy(rm_hbm.at[pl.ds(v0 * 3, 3 * NB1 * Q1)],
                        rm_v.at[pl.ds(0, 3 * NB1 * Q1)])

    @pl.when(last)
    def _():
        pltpu.sync_copy(rm_hbm.at[pl.ds(v0 * 3, 3 * NB1 * TAIL1)],
                        rm_v.at[pl.ds(0, 3 * NB1 * TAIL1)])

    def fire_cv(b, slot):
        idx = rm_v.at[pl.ds(b * 3 * NB1, 3 * NB1)]
        pltpu.async_copy(ucol_hbm.at[idx], cols_v.at[slot], csem.at[slot])
        pltpu.async_copy(uval_hbm.at[idx], vals_v.at[slot], csem.at[slot])

    def wait_cv(slot):
        pltpu.make_async_copy(ucol_hbm.at[pl.ds(0, 3 * NB1)],
                              cols_v.at[slot], csem.at[slot]).wait()
        pltpu.make_async_copy(uval_hbm.at[pl.ds(0, 3 * NB1)],
                              vals_v.at[slot], csem.at[slot]).wait()

    def fire_rows(slot):
        pltpu.async_copy(x_hbm.at[cols_v.at[slot]], rows_v.at[slot],
                         gsem.at[slot])

    def wait_rows(slot):
        pltpu.make_async_copy(x_hbm.at[pl.ds(0, 3 * NB1)],
                              rows_v.at[slot], gsem.at[slot]).wait()

    def wait_out(slot):
        pltpu.make_async_copy(out_v.at[slot],
                              out_hbm.at[pl.ds(0, NB1)], osem.at[slot]).wait()

    @pl.when(nb >= 1)
    def _():
        fire_cv(0, 0)

    @pl.when(nb >= 2)
    def _():
        fire_cv(1, 1)

    @pl.when(nb >= 1)
    def _():
        wait_cv(0)
        fire_rows(0)

    def step(b2, carry):
        b = 2 * b2
        for slot in (0, 1):
            bb = b + slot
            wait_rows(slot)

            @pl.when(bb + 1 < nb)
            def _():
                wait_cv(1 - slot)
                fire_rows(1 - slot)

            @pl.when(b2 > 0)
            def _():
                wait_out(slot)

            _pool_compute(vals_v.at[slot], rows_v.at[slot], out_v.at[slot])

            @pl.when(bb + 2 < nb)
            def _():
                fire_cv(bb + 2, slot)

            pltpu.async_copy(out_v.at[slot],
                             out_hbm.at[pl.ds(v0 + bb * NB1, NB1)],
                             osem.at[slot])
        return carry

    lax.fori_loop(0, nb // 2, step, 0)

    @pl.when(nb >= 1)
    def _():
        wait_out(0)

    @pl.when(nb >= 2)
    def _():
        wait_out(1)


@functools.partial(
    pl.kernel,
    out_type=jax.ShapeDtypeStruct((COV1, C), jnp.float32),
    mesh=_mesh,
    scratch_types=[
        pltpu.VMEM((3 * NB1 * max(P1, Q1),), jnp.int32),
        pltpu.VMEM((2, 3 * NB1), jnp.int32),
        pltpu.VMEM((2, 3 * NB1), jnp.float32),
        pltpu.VMEM((2, 3 * NB1, C), jnp.float32),
        pltpu.VMEM((2, NB1, C), jnp.float32),
        pltpu.SemaphoreType.DMA((2,)),
        pltpu.SemaphoreType.DMA((2,)),
        pltpu.SemaphoreType.DMA((2,)),
    ],
)
def _pool_kernel(x_hbm, rm_hbm, ucol_hbm, uval_hbm, out_hbm,
                 rm_v, cols_v, vals_v, rows_v, out_v, csem, gsem, osem):
    _pool_body(x_hbm, rm_hbm, ucol_hbm, uval_hbm, out_hbm,
               rm_v, cols_v, vals_v, rows_v, out_v, csem, gsem, osem)


def _dw_compute(wdw_v, rows_v, out_v):
    for k in range(NCK):
        w = [wdw_v[s, pl.ds(L * k, L)] for s in range(S)]

        def vert(ii, c2):
            for u in range(2):
                i = 2 * ii + u
                acc = rows_v[S * i, pl.ds(L * k, L)] * w[0]
                for s in range(1, S):
                    acc = acc + rows_v[S * i + s, pl.ds(L * k, L)] * w[s]
                out_v[i, pl.ds(L * k, L)] = acc
            return c2

        lax.fori_loop(0, NB2 // 2, vert, 0)


def _dw_body(pooled_hbm, sidx_hbm, wdw_hbm, out_hbm,
             sidx_v, rows_v, out_v, wdw_v, gsem, osem):
    s = lax.axis_index("s")
    c = lax.axis_index("c")
    v0 = (s * (P2 + Q2) + c * P2) * NB2
    nb = jnp.where(c == 0, P2, Q2)
    pltpu.sync_copy(wdw_hbm, wdw_v)

    @pl.when(c == 0)
    def _():
        pltpu.sync_copy(sidx_hbm.at[pl.ds(v0 * S, S * NB2 * P2)],
                        sidx_v.at[pl.ds(0, S * NB2 * P2)])

    @pl.when(c == 1)
    def _():
        pltpu.sync_copy(sidx_hbm.at[pl.ds(v0 * S, S * NB2 * Q2)],
                        sidx_v.at[pl.ds(0, S * NB2 * Q2)])

    def fire(b, slot):
        for t in range(3):
            pltpu.async_copy(
                pooled_hbm.at[sidx_v.at[pl.ds(b * S * NB2 + 96 * t, 96)]],
                rows_v.at[slot, pl.ds(96 * t, 96)], gsem.at[slot])

    def wait_in(slot):
        pltpu.make_async_copy(pooled_hbm.at[pl.ds(0, S * NB2)],
                              rows_v.at[slot], gsem.at[slot]).wait()

    def wait_out(slot):
        pltpu.make_async_copy(out_v.at[slot],
                              out_hbm.at[pl.ds(0, NB2)], osem.at[slot]).wait()

    fire(0, 0)

    def step(b2, carry):
        b = 2 * b2
        for slot in (0, 1):
            bb = b + slot
            wait_in(slot)

            @pl.when(bb + 1 < nb)
            def _():
                fire(bb + 1, 1 - slot)

            @pl.when(b2 > 0)
            def _():
                wait_out(slot)

            _dw_compute(wdw_v, rows_v.at[slot], out_v.at[slot])
            pltpu.async_copy(out_v.at[slot],
                             out_hbm.at[pl.ds(v0 + bb * NB2, NB2)],
                             osem.at[slot])
        return carry

    lax.fori_loop(0, nb // 2, step, 0)
    wait_out(0)
    wait_out(1)


@functools.partial(
    pl.kernel,
    out_type=jax.ShapeDtypeStruct((N_PAD2, C), jnp.float32),
    mesh=_mesh,
    scratch_types=[
        pltpu.VMEM((S * NB2 * max(P2, Q2),), jnp.int32),
        pltpu.VMEM((2, S * NB2, C), jnp.float32),
        pltpu.VMEM((2, NB2, C), jnp.float32),
        pltpu.VMEM((S, C), jnp.float32),
        pltpu.SemaphoreType.DMA((2,)),
        pltpu.SemaphoreType.DMA((2,)),
    ],
)
def _dw_kernel(pooled_hbm, sidx_hbm, wdw_hbm, out_hbm,
               sidx_v, rows_v, out_v, wdw_v, gsem, osem):
    _dw_body(pooled_hbm, sidx_hbm, wdw_hbm, out_hbm,
             sidx_v, rows_v, out_v, wdw_v, gsem, osem)


MM_BLK = 400


def _mm_body(dw_ref, wpw_ref, bdw_ref, bpw_ref, o_ref):
    a = dw_ref[...] + bdw_ref[...]
    acc = jnp.dot(a, wpw_ref[...], preferred_element_type=jnp.float32)
    o_ref[...] = jnp.maximum(acc + bpw_ref[...], 0.0)


def _pointwise(dw, w_pw, b_dw, b_pw):
    return pl.pallas_call(
        _mm_body,
        grid=(N_OUT // MM_BLK,),
        in_specs=[
            pl.BlockSpec((MM_BLK, C), lambda b: (b, 0)),
            pl.BlockSpec((C, C), lambda b: (0, 0)),
            pl.BlockSpec((1, C), lambda b: (0, 0)),
            pl.BlockSpec((1, C), lambda b: (0, 0)),
        ],
        out_specs=pl.BlockSpec((MM_BLK, C), lambda b: (b, 0)),
        out_shape=jax.ShapeDtypeStruct((N_OUT, C), jnp.float32),
    )(dw, w_pw, b_dw.reshape(1, C), b_pw.reshape(1, C))


def kernel(x, up_row, up_col, up_value, row_map, indices, w_dw, b_dw, w_pw, b_pw):
    del up_row
    x2 = x.reshape(N_IN, C)
    rm_flat = row_map.astype(jnp.int32).reshape(-1)
    sidx_flat = jnp.pad(indices.astype(jnp.int32),
                        ((0, N_PAD2 - N_OUT), (0, 0))).reshape(-1)
    wdw_t = w_dw.T

    pooled = _pool_kernel(x2, rm_flat, up_col.astype(jnp.int32), up_value)
    dw = _dw_kernel(pooled, sidx_flat, wdw_t)
    pw = _pointwise(dw, w_pw, b_dw, b_pw)
    return pw.reshape(1, N_OUT, C)
